# baseline (speedup 1.0000x reference)
import jax
import jax.numpy as jnp
from jax.experimental import pallas as pl
from jax.experimental.pallas import tpu as pltpu


def _make_conv_kernel(cin, cout, hw, nb, dr):
    w = hw
    wo = hw // 2

    def _conv_kernel(a_ref, w_ref, b_ref, o_ref, s5_ref):
        s = pl.program_id(1)

        @pl.when(s == 0)
        def _():
            for dx in range(5):
                s5_ref[dx] = a_ref[:, :, dx:dx + w, :]

        h0 = 2 * dr * s
        bias = b_ref[...][:, :, None]
        prev = None
        for j in range(2 * dr):
            pj = s5_ref[:, :, pl.ds(h0 + j, 5), :, :]
            c = jnp.dot(w_ref[...], pj.reshape(25 * cin, w * nb),
                        preferred_element_type=jnp.float32)
            if j % 2 == 0:
                prev = c
            else:
                m = jnp.maximum(prev, c).reshape(cout, wo, 2, nb)
                m = jnp.maximum(m[:, :, 0, :], m[:, :, 1, :])
                o_ref[:, j // 2, :, :] = (m + bias).astype(o_ref.dtype)

    return _conv_kernel


def _conv_pool(a, wm, b, cin, cout, hw, nb, dr):
    N = a.shape[-1]
    wo = hw // 2
    grid = (N // nb, wo // dr)
    return pl.pallas_call(
        _make_conv_kernel(cin, cout, hw, nb, dr),
        out_shape=jax.ShapeDtypeStruct((cout, wo, wo, N), jnp.bfloat16),
        grid=grid,
        in_specs=[
            pl.BlockSpec((cin, hw + 4, hw + 4, nb), lambda i, s: (0, 0, 0, i)),
            pl.BlockSpec((cout, 25 * cin), lambda i, s: (0, 0)),
            pl.BlockSpec((cout, 1), lambda i, s: (0, 0)),
        ],
        out_specs=pl.BlockSpec((cout, dr, wo, nb), lambda i, s: (0, s, 0, i)),
        scratch_shapes=[
            pltpu.VMEM((5, cin, hw + 4, hw, nb), jnp.bfloat16),
        ],
        compiler_params=pltpu.CompilerParams(
            dimension_semantics=("parallel", "arbitrary"),
            vmem_limit_bytes=100 * 1024 * 1024,
        ),
    )(a, wm, b)


def _fc_kernel(x_ref, w1_ref, b1_ref, w2_ref, b2_ref, o_ref):
    h = jnp.dot(w1_ref[...], x_ref[...], preferred_element_type=jnp.float32)
    h = (h + b1_ref[...]).astype(jnp.bfloat16)
    o = jnp.dot(w2_ref[...], h, preferred_element_type=jnp.float32)
    o_ref[...] = o + b2_ref[...]


def _prep_conv_w(w):
    return jnp.transpose(w, (0, 3, 1, 2)).reshape(w.shape[0], -1).astype(
        jnp.bfloat16)


def _pad_hw(a):
    return jnp.pad(a, ((0, 0), (2, 2), (2, 2), (0, 0)))


def kernel(x, w1, b1, w2, b2, w3, b3, wf1, bf1, wf2, bf2):
    N = x.shape[0]
    nb = 128 if N % 128 == 0 else N
    f32 = jnp.float32
    xt = _pad_hw(jnp.transpose(x, (1, 2, 3, 0)).astype(jnp.bfloat16))
    p1 = _conv_pool(xt, _prep_conv_w(w1), b1.reshape(32, 1).astype(f32),
                    3, 32, 32, nb, 16)
    p2 = _conv_pool(_pad_hw(p1), _prep_conv_w(w2),
                    b2.reshape(32, 1).astype(f32), 32, 32, 16, nb, 4)
    p3 = _conv_pool(_pad_hw(p2), _prep_conv_w(w3),
                    b3.reshape(64, 1).astype(f32), 32, 64, 8, nb, 4)
    flat = p3.reshape(1024, N)
    out = pl.pallas_call(
        _fc_kernel,
        out_shape=jax.ShapeDtypeStruct((10, N), f32),
        grid=(N // nb,),
        in_specs=[
            pl.BlockSpec((1024, nb), lambda i: (0, i)),
            pl.BlockSpec((64, 1024), lambda i: (0, 0)),
            pl.BlockSpec((64, 1), lambda i: (0, 0)),
            pl.BlockSpec((10, 64), lambda i: (0, 0)),
            pl.BlockSpec((10, 1), lambda i: (0, 0)),
        ],
        out_specs=pl.BlockSpec((10, nb), lambda i: (0, i)),
        compiler_params=pltpu.CompilerParams(
            dimension_semantics=("parallel",),
        ),
    )(flat, wf1.astype(jnp.bfloat16), bf1.reshape(64, 1).astype(f32),
      wf2.astype(jnp.bfloat16), bf2.reshape(10, 1).astype(f32))
    return jnp.transpose(out)


# speedup vs baseline: 1.0423x; 1.0423x over previous
import jax
import jax.numpy as jnp
from jax.experimental import pallas as pl
from jax.experimental.pallas import tpu as pltpu


def _make_conv_kernel(cin, cout, hw, nb, dr):
    w = hw
    wo = hw // 2

    def _conv_kernel(a_ref, w_ref, b_ref, o_ref, s5_ref):
        s = pl.program_id(1)

        @pl.when(s == 0)
        def _():
            for dx in range(5):
                s5_ref[dx] = a_ref[:, :, dx:dx + w, :]

        h0 = 2 * dr * s
        bias = b_ref[...][:, :, None]
        prev = None
        for j in range(2 * dr):
            pj = s5_ref[:, :, pl.ds(h0 + j, 5), :, :]
            c = jnp.dot(w_ref[...], pj.reshape(25 * cin, w * nb),
                        preferred_element_type=jnp.float32)
            if j % 2 == 0:
                prev = c
            else:
                m = jnp.maximum(prev, c).reshape(cout, wo, 2, nb)
                m = jnp.maximum(m[:, :, 0, :], m[:, :, 1, :])
                o_ref[:, j // 2, :, :] = (m + bias).astype(o_ref.dtype)

    return _conv_kernel


def _conv_pool(a, wm, b, cin, cout, hw, nb, dr):
    N = a.shape[-1]
    wo = hw // 2
    grid = (N // nb, wo // dr)
    return pl.pallas_call(
        _make_conv_kernel(cin, cout, hw, nb, dr),
        out_shape=jax.ShapeDtypeStruct((cout, wo, wo, N), jnp.bfloat16),
        grid=grid,
        in_specs=[
            pl.BlockSpec((cin, hw + 4, hw + 4, nb), lambda i, s: (0, 0, 0, i)),
            pl.BlockSpec((cout, 25 * cin), lambda i, s: (0, 0)),
            pl.BlockSpec((cout, 1), lambda i, s: (0, 0)),
        ],
        out_specs=pl.BlockSpec((cout, dr, wo, nb), lambda i, s: (0, s, 0, i)),
        scratch_shapes=[
            pltpu.VMEM((5, cin, hw + 4, hw, nb), jnp.bfloat16),
        ],
        compiler_params=pltpu.CompilerParams(
            dimension_semantics=("parallel", "arbitrary"),
            vmem_limit_bytes=100 * 1024 * 1024,
        ),
    )(a, wm, b)


def _fc_kernel(x_ref, w1_ref, b1_ref, w2_ref, b2_ref, o_ref):
    h = jnp.dot(w1_ref[...], x_ref[...], preferred_element_type=jnp.float32)
    h = (h + b1_ref[...]).astype(jnp.bfloat16)
    o = jnp.dot(w2_ref[...], h, preferred_element_type=jnp.float32)
    o_ref[...] = o + b2_ref[...]


def _prep_conv_w(w):
    return jnp.transpose(w, (0, 3, 1, 2)).reshape(w.shape[0], -1).astype(
        jnp.bfloat16)


def _pad_hw(a):
    return jnp.pad(a, ((0, 0), (2, 2), (2, 2), (0, 0)))


def kernel(x, w1, b1, w2, b2, w3, b3, wf1, bf1, wf2, bf2):
    N = x.shape[0]
    nb = 128 if N % 128 == 0 else N
    f32 = jnp.float32
    xt = _pad_hw(jnp.transpose(x, (1, 2, 3, 0)).astype(jnp.bfloat16))
    p1 = _conv_pool(xt, _prep_conv_w(w1), b1.reshape(32, 1).astype(f32),
                    3, 32, 32, nb, 8)
    p2 = _conv_pool(_pad_hw(p1), _prep_conv_w(w2),
                    b2.reshape(32, 1).astype(f32), 32, 32, 16, nb, 2)
    p3 = _conv_pool(_pad_hw(p2), _prep_conv_w(w3),
                    b3.reshape(64, 1).astype(f32), 32, 64, 8, nb, 2)
    flat = p3.reshape(1024, N)
    out = pl.pallas_call(
        _fc_kernel,
        out_shape=jax.ShapeDtypeStruct((10, N), f32),
        grid=(N // nb,),
        in_specs=[
            pl.BlockSpec((1024, nb), lambda i: (0, i)),
            pl.BlockSpec((64, 1024), lambda i: (0, 0)),
            pl.BlockSpec((64, 1), lambda i: (0, 0)),
            pl.BlockSpec((10, 64), lambda i: (0, 0)),
            pl.BlockSpec((10, 1), lambda i: (0, 0)),
        ],
        out_specs=pl.BlockSpec((10, nb), lambda i: (0, i)),
        compiler_params=pltpu.CompilerParams(
            dimension_semantics=("parallel",),
        ),
    )(flat, wf1.astype(jnp.bfloat16), bf1.reshape(64, 1).astype(f32),
      wf2.astype(jnp.bfloat16), bf2.reshape(10, 1).astype(f32))
    return jnp.transpose(out)
